# Initial kernel scaffold; baseline (speedup 1.0000x reference)
import jax
import jax.numpy as jnp
from jax.experimental import pallas as pl


def kernel(user_embed, item_embed, edge_values, edge_index, users, pos_items, neg_items, cur_epoch):
    raise NotImplementedError("write your pallas kernel here")


# speedup vs baseline: 2.0774x; 2.0774x over previous
import functools

import jax
import jax.numpy as jnp
from jax import lax
from jax.experimental import pallas as pl
from jax.experimental.pallas import tpu as pltpu
from jax.experimental.pallas import tpu_sc as plsc

N_U = 25000
N_I = 25000
N = N_U + N_I
D = 64
E = 800000
HOPS = 3
B = 4096
DECAY = 1e-4

NCORES = 2
NSUB = 16
CH = 128
EP = 800768
ECS = EP // NSUB
NCHUNK = ECS // CH
TAB = 25088
NP = 2 * TAB
STRIPE = TAB // NSUB
ZB = 128

_mesh = plsc.VectorSubcoreMesh(core_axis_name="c", subcore_axis_name="s")


@functools.partial(
    pl.kernel,
    out_type=jax.ShapeDtypeStruct((NP, D), jnp.float32),
    mesh=_mesh,
    compiler_params=pltpu.CompilerParams(use_tc_tiling_on_sc=False),
    scratch_types=[
        pltpu.VMEM_SHARED((TAB, D), jnp.float32),
        pltpu.VMEM((CH,), jnp.int32),
        pltpu.VMEM((CH,), jnp.int32),
        pltpu.VMEM((CH,), jnp.float32),
        pltpu.VMEM((CH, D), jnp.float32),
        pltpu.VMEM((ZB, D), jnp.float32),
        pltpu.SemaphoreType.DMA,
    ],
)
def _hop(agg_hbm, rows_hbm, cols_hbm, vals_hbm, out_hbm,
         table, colv, rowv, valv, gath, zbuf, sem):
    c = lax.axis_index("c")
    s = lax.axis_index("s")

    z16 = jnp.zeros((16,), jnp.float32)

    def _zrow(i, carry):
        for j in range(D // 16):
            zbuf[i, pl.ds(j * 16, 16)] = z16
        return carry

    lax.fori_loop(0, ZB, _zrow, 0)

    zbase = s * STRIPE
    nfull = STRIPE // ZB
    rem = STRIPE - nfull * ZB

    def _zcp(i, carry):
        pltpu.sync_copy(zbuf, table.at[pl.ds(zbase + i * ZB, ZB)])
        return carry

    lax.fori_loop(0, nfull, _zcp, 0)
    if rem:
        pltpu.sync_copy(zbuf.at[pl.ds(0, rem)],
                        table.at[pl.ds(zbase + nfull * ZB, rem)])
    plsc.subcore_barrier()

    row_base = c * N_U
    e0 = s * ECS

    def _chunk(i, carry):
        off = e0 + i * CH
        pltpu.sync_copy(rows_hbm.at[pl.ds(off, CH)], rowv)
        pltpu.sync_copy(cols_hbm.at[pl.ds(off, CH)], colv)
        pltpu.sync_copy(vals_hbm.at[pl.ds(off, CH)], valv)
        for g in range(CH // 16):
            sl = pl.ds(g * 16, 16)
            cc = colv[sl]
            colv[sl] = jnp.where(cc >= N_U, cc + (TAB - N_U), cc)
        pltpu.async_copy(agg_hbm.at[colv], gath, sem).wait()
        for g in range(CH // 16):
            sl = pl.ds(g * 16, 16)
            rl = rowv[sl] - row_base
            ok = (rl >= 0) & (rl < N_U)
            rowv[sl] = jnp.where(ok, rl, 0)
            valv[sl] = jnp.where(ok, valv[sl], 0.0)

        def _mgrp(g, carry2):
            vvec = valv[pl.ds(g * 16, 16)]
            for l in range(16):
                vv = jnp.full((16,), vvec[l], jnp.float32)
                e = g * 16 + l
                for j in range(D // 16):
                    sl = pl.ds(j * 16, 16)
                    gath[e, sl] = gath[e, sl] * vv
            return carry2

        lax.fori_loop(0, CH // 16, _mgrp, 0)
        pltpu.sync_copy(gath, table.at[rowv], add=True)
        return carry

    lax.fori_loop(0, NCHUNK, _chunk, 0)
    plsc.subcore_barrier()

    pltpu.sync_copy(table.at[pl.ds(zbase, STRIPE)],
                    out_hbm.at[pl.ds(c * TAB + zbase, STRIPE)])


_tail_out = [jax.ShapeDtypeStruct((B, D), jnp.float32)] * 6


@functools.partial(
    pl.kernel,
    out_type=_tail_out,
    mesh=_mesh,
    compiler_params=pltpu.CompilerParams(use_tc_tiling_on_sc=False),
    scratch_types=[
        pltpu.VMEM((CH,), jnp.int32),
        pltpu.VMEM((CH, D), jnp.float32),
        pltpu.VMEM((CH, D), jnp.float32),
        pltpu.SemaphoreType.DMA,
    ],
)
def _tail(t0, t1, t2, t3, uid, pid, nid,
          ue_o, pe_o, ne_o, u0_o, p0_o, n0_o,
          idxv, gath, acc, sem):
    c = lax.axis_index("c")
    s = lax.axis_index("s")
    w = s * NCORES + c
    base = w * CH

    for idx_hbm, mean_o, h0_o in ((uid, ue_o, u0_o),
                                  (pid, pe_o, p0_o),
                                  (nid, ne_o, n0_o)):
        pltpu.sync_copy(idx_hbm.at[pl.ds(base, CH)], idxv)
        pltpu.async_copy(t0.at[idxv], gath, sem).wait()
        pltpu.sync_copy(gath, h0_o.at[pl.ds(base, CH)])

        def _cp(e, carry):
            for j in range(D // 16):
                sl = pl.ds(j * 16, 16)
                acc[e, sl] = gath[e, sl]
            return carry

        lax.fori_loop(0, CH, _cp, 0)

        for t in (t1, t2, t3):
            pltpu.async_copy(t.at[idxv], gath, sem).wait()

            def _acc(e, carry):
                for j in range(D // 16):
                    sl = pl.ds(j * 16, 16)
                    acc[e, sl] = acc[e, sl] + gath[e, sl]
                return carry

            lax.fori_loop(0, CH, _acc, 0)

        quarter = jnp.full((16,), 0.25, jnp.float32)

        def _scale(e, carry):
            for j in range(D // 16):
                sl = pl.ds(j * 16, 16)
                acc[e, sl] = acc[e, sl] * quarter
            return carry

        lax.fori_loop(0, CH, _scale, 0)
        pltpu.sync_copy(acc, mean_o.at[pl.ds(base, CH)])


def _loss_body(ue_ref, pe_ref, ne_ref, u0_ref, p0_ref, n0_ref, out_ref):
    ue = ue_ref[...]
    pos_s = jnp.sum(ue * pe_ref[...], axis=1, keepdims=True)
    neg_s = jnp.sum(ue * ne_ref[...], axis=1, keepdims=True)
    mf = jnp.mean(jnp.log(1.0 + jnp.exp(neg_s - pos_s)))
    reg = (jnp.sum(u0_ref[...] ** 2) + jnp.sum(p0_ref[...] ** 2)
           + jnp.sum(n0_ref[...] ** 2)) * 0.5
    emb = DECAY * reg / B
    lanes = lax.broadcasted_iota(jnp.int32, (1, 128), 1)
    out_ref[...] = jnp.where(
        lanes == 0, mf + emb,
        jnp.where(lanes == 1, mf, jnp.where(lanes == 2, emb, 0.0)))


_loss = pl.pallas_call(
    _loss_body,
    out_shape=jax.ShapeDtypeStruct((1, 128), jnp.float32),
)


def kernel(user_embed, item_embed, edge_values, edge_index, users,
           pos_items, neg_items, cur_epoch):
    zpad = jnp.zeros((TAB - N_U, D), jnp.float32)
    all0 = jnp.concatenate([user_embed, zpad, item_embed, zpad], axis=0)
    pad = EP - E
    rows_p = jnp.pad(edge_index[0], (0, pad))
    cols_p = jnp.pad(edge_index[1], (0, pad))
    vals_p = jnp.pad(edge_values, (0, pad))

    a1 = _hop(all0, rows_p, cols_p, vals_p)
    a2 = _hop(a1, rows_p, cols_p, vals_p)
    a3 = _hop(a2, rows_p, cols_p, vals_p)

    uid = users
    pid = pos_items + TAB
    nid = neg_items[:, 0] + TAB
    ue, pe, ne, u0, p0, n0 = _tail(all0, a1, a2, a3, uid, pid, nid)

    out = _loss(ue, pe, ne, u0, p0, n0)
    return (out[0, 0], out[0, 1], out[0, 2])


# speedup vs baseline: 3.5727x; 1.7198x over previous
import functools

import jax
import jax.numpy as jnp
from jax import lax
from jax.experimental import pallas as pl
from jax.experimental.pallas import tpu as pltpu
from jax.experimental.pallas import tpu_sc as plsc

N_U = 25000
N_I = 25000
N = N_U + N_I
D = 64
E = 800000
HOPS = 3
B = 4096
DECAY = 1e-4

NCORES = 2
NSUB = 16
CH = 128
NCHUNK = 392
EP = NSUB * NCHUNK * CH
ECS = EP // NSUB
NIDX = 8
NGB = 2
TAB = 25088
NP = 2 * TAB
STRIPE = TAB // NSUB
ZB = 32

_mesh = plsc.VectorSubcoreMesh(core_axis_name="c", subcore_axis_name="s")


@functools.partial(
    pl.kernel,
    out_type=jax.ShapeDtypeStruct((NP, D), jnp.float32),
    mesh=_mesh,
    compiler_params=pltpu.CompilerParams(use_tc_tiling_on_sc=False),
    scratch_types=[
        pltpu.VMEM_SHARED((TAB, D), jnp.float32),
        pltpu.VMEM((NIDX, CH), jnp.int32),
        pltpu.VMEM((NIDX, CH), jnp.int32),
        pltpu.VMEM((NIDX, CH), jnp.float32),
        pltpu.VMEM((NGB, CH, D), jnp.float32),
        pltpu.VMEM((ZB, D), jnp.float32),
        [pltpu.SemaphoreType.DMA] * NIDX,
        [pltpu.SemaphoreType.DMA] * NGB,
        [pltpu.SemaphoreType.DMA] * NGB,
    ],
)
def _hop(agg_hbm, rows_hbm, cols_hbm, vals_hbm, out_hbm,
         table, colv, rowv, valv, gath, zbuf, isem, gsem, ssem):
    c = lax.axis_index("c")
    s = lax.axis_index("s")

    z16 = jnp.zeros((16,), jnp.float32)

    def _zrow(i, carry):
        for j in range(D // 16):
            zbuf[i, pl.ds(j * 16, 16)] = z16
        return carry

    lax.fori_loop(0, ZB, _zrow, 0)

    zbase = s * STRIPE
    nfull = STRIPE // ZB
    rem = STRIPE - nfull * ZB

    for q in range(nfull):
        pltpu.async_copy(zbuf, table.at[pl.ds(zbase + q * ZB, ZB)], gsem[0])
    if rem:
        pltpu.async_copy(zbuf.at[pl.ds(0, rem)],
                         table.at[pl.ds(zbase + nfull * ZB, rem)], gsem[0])
    for q in range(nfull):
        pltpu.make_async_copy(zbuf, table.at[pl.ds(zbase, ZB)], gsem[0]).wait()
    if rem:
        pltpu.make_async_copy(zbuf.at[pl.ds(0, rem)],
                              table.at[pl.ds(zbase, rem)], gsem[0]).wait()
    plsc.subcore_barrier()

    row_base = c * N_U
    e0 = s * ECS

    def p1(i, k):
        o = e0 + i * CH
        pltpu.async_copy(rows_hbm.at[pl.ds(o, CH)], rowv.at[k], isem[k])
        pltpu.async_copy(cols_hbm.at[pl.ds(o, CH)], colv.at[k], isem[k])
        pltpu.async_copy(vals_hbm.at[pl.ds(o, CH)], valv.at[k], isem[k])

    def p2(i, k8, k4, sswait):
        o = e0 + i * CH
        pltpu.make_async_copy(rows_hbm.at[pl.ds(o, CH)], rowv.at[k8],
                              isem[k8]).wait()
        pltpu.make_async_copy(cols_hbm.at[pl.ds(o, CH)], colv.at[k8],
                              isem[k8]).wait()
        pltpu.make_async_copy(vals_hbm.at[pl.ds(o, CH)], valv.at[k8],
                              isem[k8]).wait()
        for g in range(CH // 16):
            sl = pl.ds(g * 16, 16)
            cc = colv[k8, sl]
            colv[k8, sl] = jnp.where(cc >= N_U, cc + (TAB - N_U), cc)
        if sswait:
            pltpu.make_async_copy(gath.at[k4], table.at[rowv.at[k8]],
                                  ssem[k4]).wait()
        pltpu.async_copy(agg_hbm.at[colv.at[k8]], gath.at[k4], gsem[k4])

    def p3(i, k8, k4):
        pltpu.make_async_copy(agg_hbm.at[colv.at[k8]], gath.at[k4],
                              gsem[k4]).wait()

        def _mgrp(g, carry):
            sl = pl.ds(g * 16, 16)
            rl = rowv[k8, sl] - row_base
            ok = (rl >= 0) & (rl < N_U)
            rowv[k8, sl] = jnp.where(ok, rl, 0)
            vv = jnp.where(ok, valv[k8, sl], 0.0)
            for l in range(16):
                vb = jnp.full((16,), vv[l], jnp.float32)
                e = g * 16 + l
                for j in range(D // 16):
                    slj = pl.ds(j * 16, 16)
                    gath[k4, e, slj] = gath[k4, e, slj] * vb
            return carry

        lax.fori_loop(0, CH // 16, _mgrp, 0)
        pltpu.async_copy(gath.at[k4], table.at[rowv.at[k8]], ssem[k4],
                         add=True)

    def step(i, kk):
        p2(i + 1, (kk + 1) % NIDX, (kk + 1) % NGB, True)
        p3(i, kk, kk % NGB)
        p1(i + 2, (kk + 2) % NIDX)

    p1(0, 0)
    p1(1, 1)
    p2(0, 0, 0, False)
    for i in range(NIDX):
        p2(i + 1, (i + 1) % NIDX, (i + 1) % NGB, (i + 1) >= NGB)
        p3(i, i, i % NGB)
        p1(i + 2, (i + 2) % NIDX)

    def _octet(t, carry):
        ib = t * NIDX
        for kk in range(NIDX):
            step(ib + kk, kk)
        return carry

    lax.fori_loop(1, NCHUNK // NIDX - 1, _octet, 0)

    base = NCHUNK - NIDX
    for kk in range(NIDX):
        i = base + kk
        if i + 1 < NCHUNK:
            p2(i + 1, (kk + 1) % NIDX, (kk + 1) % NGB, True)
        p3(i, kk, kk % NGB)
        if i + 2 < NCHUNK:
            p1(i + 2, (kk + 2) % NIDX)
    for k in range(NGB):
        k8 = (NCHUNK - NGB + k) % NIDX
        pltpu.make_async_copy(gath.at[k], table.at[rowv.at[k8]],
                              ssem[k]).wait()
    plsc.subcore_barrier()

    pltpu.sync_copy(table.at[pl.ds(zbase, STRIPE)],
                    out_hbm.at[pl.ds(c * TAB + zbase, STRIPE)])


_tail_out = [jax.ShapeDtypeStruct((B, D), jnp.float32)] * 6


@functools.partial(
    pl.kernel,
    out_type=_tail_out,
    mesh=_mesh,
    compiler_params=pltpu.CompilerParams(use_tc_tiling_on_sc=False),
    scratch_types=[
        pltpu.VMEM((CH,), jnp.int32),
        pltpu.VMEM((CH, D), jnp.float32),
        pltpu.VMEM((CH, D), jnp.float32),
        pltpu.SemaphoreType.DMA,
    ],
)
def _tail(t0, t1, t2, t3, uid, pid, nid,
          ue_o, pe_o, ne_o, u0_o, p0_o, n0_o,
          idxv, gath, acc, sem):
    c = lax.axis_index("c")
    s = lax.axis_index("s")
    w = s * NCORES + c
    base = w * CH

    for idx_hbm, mean_o, h0_o in ((uid, ue_o, u0_o),
                                  (pid, pe_o, p0_o),
                                  (nid, ne_o, n0_o)):
        pltpu.sync_copy(idx_hbm.at[pl.ds(base, CH)], idxv)
        pltpu.async_copy(t0.at[idxv], gath, sem).wait()
        pltpu.sync_copy(gath, h0_o.at[pl.ds(base, CH)])

        def _cp(e, carry):
            for j in range(D // 16):
                sl = pl.ds(j * 16, 16)
                acc[e, sl] = gath[e, sl]
            return carry

        lax.fori_loop(0, CH, _cp, 0)

        for t in (t1, t2, t3):
            pltpu.async_copy(t.at[idxv], gath, sem).wait()

            def _acc(e, carry):
                for j in range(D // 16):
                    sl = pl.ds(j * 16, 16)
                    acc[e, sl] = acc[e, sl] + gath[e, sl]
                return carry

            lax.fori_loop(0, CH, _acc, 0)

        quarter = jnp.full((16,), 0.25, jnp.float32)

        def _scale(e, carry):
            for j in range(D // 16):
                sl = pl.ds(j * 16, 16)
                acc[e, sl] = acc[e, sl] * quarter
            return carry

        lax.fori_loop(0, CH, _scale, 0)
        pltpu.sync_copy(acc, mean_o.at[pl.ds(base, CH)])


def _loss_body(ue_ref, pe_ref, ne_ref, u0_ref, p0_ref, n0_ref, out_ref):
    ue = ue_ref[...]
    pos_s = jnp.sum(ue * pe_ref[...], axis=1, keepdims=True)
    neg_s = jnp.sum(ue * ne_ref[...], axis=1, keepdims=True)
    mf = jnp.mean(jnp.log(1.0 + jnp.exp(neg_s - pos_s)))
    reg = (jnp.sum(u0_ref[...] ** 2) + jnp.sum(p0_ref[...] ** 2)
           + jnp.sum(n0_ref[...] ** 2)) * 0.5
    emb = DECAY * reg / B
    lanes = lax.broadcasted_iota(jnp.int32, (1, 128), 1)
    out_ref[...] = jnp.where(
        lanes == 0, mf + emb,
        jnp.where(lanes == 1, mf, jnp.where(lanes == 2, emb, 0.0)))


_loss = pl.pallas_call(
    _loss_body,
    out_shape=jax.ShapeDtypeStruct((1, 128), jnp.float32),
)


def kernel(user_embed, item_embed, edge_values, edge_index, users,
           pos_items, neg_items, cur_epoch):
    zpad = jnp.zeros((TAB - N_U, D), jnp.float32)
    all0 = jnp.concatenate([user_embed, zpad, item_embed, zpad], axis=0)
    pad = EP - E
    rows_p = jnp.pad(edge_index[0], (0, pad))
    cols_p = jnp.pad(edge_index[1], (0, pad))
    vals_p = jnp.pad(edge_values, (0, pad))

    a1 = _hop(all0, rows_p, cols_p, vals_p)
    a2 = _hop(a1, rows_p, cols_p, vals_p)
    a3 = _hop(a2, rows_p, cols_p, vals_p)

    uid = users
    pid = pos_items + TAB
    nid = neg_items[:, 0] + TAB
    ue, pe, ne, u0, p0, n0 = _tail(all0, a1, a2, a3, uid, pid, nid)

    out = _loss(ue, pe, ne, u0, p0, n0)
    return (out[0, 0], out[0, 1], out[0, 2])


# speedup vs baseline: 3.5733x; 1.0002x over previous
import functools

import jax
import jax.numpy as jnp
import numpy as np
from jax import lax
from jax.experimental import pallas as pl
from jax.experimental.pallas import tpu as pltpu
from jax.experimental.pallas import tpu_sc as plsc

N_U = 25000
N_I = 25000
N = N_U + N_I
D = 64
E = 800000
HOPS = 3
B = 4096
DECAY = 1e-4

NCORES = 2
NSUB = 16
CH = 128
NCHUNK = 392
EP = NSUB * NCHUNK * CH
ECS = EP // NSUB
NIDX = 8
NGB = 2
TAB = 25088
NP = 2 * TAB
STRIPE = TAB // NSUB
ZB = 32

_mesh = plsc.VectorSubcoreMesh(core_axis_name="c", subcore_axis_name="s")

_GDN = lax.GatherDimensionNumbers(offset_dims=(), collapsed_slice_dims=(0,),
                                  start_index_map=(0,))


@functools.partial(
    pl.kernel,
    out_type=jax.ShapeDtypeStruct((NP, D), jnp.float32),
    mesh=_mesh,
    compiler_params=pltpu.CompilerParams(use_tc_tiling_on_sc=False),
    scratch_types=[
        pltpu.VMEM_SHARED((TAB, D), jnp.float32),
        pltpu.VMEM((NIDX, CH), jnp.int32),
        pltpu.VMEM((NIDX, CH), jnp.int32),
        pltpu.VMEM((NIDX, CH), jnp.float32),
        pltpu.VMEM((NGB, CH, D), jnp.float32),
        pltpu.VMEM((ZB, D), jnp.float32),
        [pltpu.SemaphoreType.DMA] * NIDX,
        [pltpu.SemaphoreType.DMA] * NGB,
        [pltpu.SemaphoreType.DMA] * NGB,
    ],
)
def _hop(agg_hbm, rows_hbm, cols_hbm, vals_hbm, out_hbm,
         table, colv, rowv, valv, gath, zbuf, isem, gsem, ssem):
    c = lax.axis_index("c")
    s = lax.axis_index("s")

    z16 = jnp.zeros((16,), jnp.float32)

    def _zrow(i, carry):
        for j in range(D // 16):
            zbuf[i, pl.ds(j * 16, 16)] = z16
        return carry

    lax.fori_loop(0, ZB, _zrow, 0)

    zbase = s * STRIPE
    nfull = STRIPE // ZB
    rem = STRIPE - nfull * ZB

    for q in range(nfull):
        pltpu.async_copy(zbuf, table.at[pl.ds(zbase + q * ZB, ZB)], gsem[0])
    if rem:
        pltpu.async_copy(zbuf.at[pl.ds(0, rem)],
                         table.at[pl.ds(zbase + nfull * ZB, rem)], gsem[0])
    for q in range(nfull):
        pltpu.make_async_copy(zbuf, table.at[pl.ds(zbase, ZB)], gsem[0]).wait()
    if rem:
        pltpu.make_async_copy(zbuf.at[pl.ds(0, rem)],
                              table.at[pl.ds(zbase, rem)], gsem[0]).wait()
    plsc.subcore_barrier()

    row_base = c * N_U
    e0 = s * ECS

    def p1(i, k):
        o = e0 + i * CH
        pltpu.async_copy(rows_hbm.at[pl.ds(o, CH)], rowv.at[k], isem[k])
        pltpu.async_copy(cols_hbm.at[pl.ds(o, CH)], colv.at[k], isem[k])
        pltpu.async_copy(vals_hbm.at[pl.ds(o, CH)], valv.at[k], isem[k])

    def p2(i, k8, k4, sswait):
        o = e0 + i * CH
        pltpu.make_async_copy(rows_hbm.at[pl.ds(o, CH)], rowv.at[k8],
                              isem[k8]).wait()
        pltpu.make_async_copy(cols_hbm.at[pl.ds(o, CH)], colv.at[k8],
                              isem[k8]).wait()
        pltpu.make_async_copy(vals_hbm.at[pl.ds(o, CH)], valv.at[k8],
                              isem[k8]).wait()
        for g in range(CH // 16):
            sl = pl.ds(g * 16, 16)
            cc = colv[k8, sl]
            colv[k8, sl] = jnp.where(cc >= N_U, cc + (TAB - N_U), cc)
        if sswait:
            pltpu.make_async_copy(gath.at[k4], table.at[rowv.at[k8]],
                                  ssem[k4]).wait()
        pltpu.async_copy(agg_hbm.at[colv.at[k8]], gath.at[k4], gsem[k4])

    def p3(i, k8, k4):
        pltpu.make_async_copy(agg_hbm.at[colv.at[k8]], gath.at[k4],
                              gsem[k4]).wait()

        def _mgrp(g, carry):
            sl = pl.ds(g * 16, 16)
            rl = rowv[k8, sl] - row_base
            ok = (rl >= 0) & (rl < N_U)
            rowv[k8, sl] = jnp.where(ok, rl, 0)
            vv = jnp.where(ok, valv[k8, sl], 0.0)
            for l in range(16):
                vb = lax.gather(vv, jnp.full((16, 1), l, jnp.int32), _GDN, (1,),
                                mode=lax.GatherScatterMode.PROMISE_IN_BOUNDS)
                e = g * 16 + l
                for j in range(D // 16):
                    slj = pl.ds(j * 16, 16)
                    gath[k4, e, slj] = gath[k4, e, slj] * vb
            return carry

        lax.fori_loop(0, CH // 16, _mgrp, 0)
        pltpu.async_copy(gath.at[k4], table.at[rowv.at[k8]], ssem[k4],
                         add=True)

    def step(i, kk):
        p2(i + 1, (kk + 1) % NIDX, (kk + 1) % NGB, True)
        p3(i, kk, kk % NGB)
        p1(i + 2, (kk + 2) % NIDX)

    p1(0, 0)
    p1(1, 1)
    p2(0, 0, 0, False)
    for i in range(NIDX):
        p2(i + 1, (i + 1) % NIDX, (i + 1) % NGB, (i + 1) >= NGB)
        p3(i, i, i % NGB)
        p1(i + 2, (i + 2) % NIDX)

    def _octet(t, carry):
        ib = t * NIDX
        for kk in range(NIDX):
            step(ib + kk, kk)
        return carry

    lax.fori_loop(1, NCHUNK // NIDX - 1, _octet, 0)

    base = NCHUNK - NIDX
    for kk in range(NIDX):
        i = base + kk
        if i + 1 < NCHUNK:
            p2(i + 1, (kk + 1) % NIDX, (kk + 1) % NGB, True)
        p3(i, kk, kk % NGB)
        if i + 2 < NCHUNK:
            p1(i + 2, (kk + 2) % NIDX)
    for k in range(NGB):
        k8 = (NCHUNK - NGB + k) % NIDX
        pltpu.make_async_copy(gath.at[k], table.at[rowv.at[k8]],
                              ssem[k]).wait()
    plsc.subcore_barrier()

    pltpu.sync_copy(table.at[pl.ds(zbase, STRIPE)],
                    out_hbm.at[pl.ds(c * TAB + zbase, STRIPE)])


_tail_out = [jax.ShapeDtypeStruct((B, D), jnp.float32)] * 6


@functools.partial(
    pl.kernel,
    out_type=_tail_out,
    mesh=_mesh,
    compiler_params=pltpu.CompilerParams(use_tc_tiling_on_sc=False),
    scratch_types=[
        pltpu.VMEM((CH,), jnp.int32),
        pltpu.VMEM((CH, D), jnp.float32),
        pltpu.VMEM((CH, D), jnp.float32),
        pltpu.SemaphoreType.DMA,
    ],
)
def _tail(t0, t1, t2, t3, uid, pid, nid,
          ue_o, pe_o, ne_o, u0_o, p0_o, n0_o,
          idxv, gath, acc, sem):
    c = lax.axis_index("c")
    s = lax.axis_index("s")
    w = s * NCORES + c
    base = w * CH

    for idx_hbm, mean_o, h0_o in ((uid, ue_o, u0_o),
                                  (pid, pe_o, p0_o),
                                  (nid, ne_o, n0_o)):
        pltpu.sync_copy(idx_hbm.at[pl.ds(base, CH)], idxv)
        pltpu.async_copy(t0.at[idxv], gath, sem).wait()
        pltpu.sync_copy(gath, h0_o.at[pl.ds(base, CH)])

        def _cp(e, carry):
            for j in range(D // 16):
                sl = pl.ds(j * 16, 16)
                acc[e, sl] = gath[e, sl]
            return carry

        lax.fori_loop(0, CH, _cp, 0)

        for t in (t1, t2, t3):
            pltpu.async_copy(t.at[idxv], gath, sem).wait()

            def _acc(e, carry):
                for j in range(D // 16):
                    sl = pl.ds(j * 16, 16)
                    acc[e, sl] = acc[e, sl] + gath[e, sl]
                return carry

            lax.fori_loop(0, CH, _acc, 0)

        quarter = jnp.full((16,), 0.25, jnp.float32)

        def _scale(e, carry):
            for j in range(D // 16):
                sl = pl.ds(j * 16, 16)
                acc[e, sl] = acc[e, sl] * quarter
            return carry

        lax.fori_loop(0, CH, _scale, 0)
        pltpu.sync_copy(acc, mean_o.at[pl.ds(base, CH)])


def _loss_body(ue_ref, pe_ref, ne_ref, u0_ref, p0_ref, n0_ref, out_ref):
    ue = ue_ref[...]
    pos_s = jnp.sum(ue * pe_ref[...], axis=1, keepdims=True)
    neg_s = jnp.sum(ue * ne_ref[...], axis=1, keepdims=True)
    mf = jnp.mean(jnp.log(1.0 + jnp.exp(neg_s - pos_s)))
    reg = (jnp.sum(u0_ref[...] ** 2) + jnp.sum(p0_ref[...] ** 2)
           + jnp.sum(n0_ref[...] ** 2)) * 0.5
    emb = DECAY * reg / B
    lanes = lax.broadcasted_iota(jnp.int32, (1, 128), 1)
    out_ref[...] = jnp.where(
        lanes == 0, mf + emb,
        jnp.where(lanes == 1, mf, jnp.where(lanes == 2, emb, 0.0)))


_loss = pl.pallas_call(
    _loss_body,
    out_shape=jax.ShapeDtypeStruct((1, 128), jnp.float32),
)


def kernel(user_embed, item_embed, edge_values, edge_index, users,
           pos_items, neg_items, cur_epoch):
    zpad = jnp.zeros((TAB - N_U, D), jnp.float32)
    all0 = jnp.concatenate([user_embed, zpad, item_embed, zpad], axis=0)
    pad = EP - E
    rows_p = jnp.pad(edge_index[0], (0, pad))
    cols_p = jnp.pad(edge_index[1], (0, pad))
    vals_p = jnp.pad(edge_values, (0, pad))

    a1 = _hop(all0, rows_p, cols_p, vals_p)
    a2 = _hop(a1, rows_p, cols_p, vals_p)
    a3 = _hop(a2, rows_p, cols_p, vals_p)

    uid = users
    pid = pos_items + TAB
    nid = neg_items[:, 0] + TAB
    ue, pe, ne, u0, p0, n0 = _tail(all0, a1, a2, a3, uid, pid, nid)

    out = _loss(ue, pe, ne, u0, p0, n0)
    return (out[0, 0], out[0, 1], out[0, 2])


# speedup vs baseline: 5.8199x; 1.6287x over previous
import functools

import jax
import jax.numpy as jnp
import numpy as np
from jax import lax
from jax.experimental import pallas as pl
from jax.experimental.pallas import tpu as pltpu
from jax.experimental.pallas import tpu_sc as plsc

N_U = 25000
N_I = 25000
N = N_U + N_I
D = 64
E = 800000
HOPS = 3
B = 4096
DECAY = 1e-4

NCORES = 2
NSUB = 16
CH = 128
NCHUNK = 392
EP = NSUB * NCHUNK * CH
ECS = EP // NSUB
NIDX = 8
NGB = 2
TAB = 25088
NP = 2 * TAB
STRIPE = TAB // NSUB
ZB = 32

_mesh = plsc.VectorSubcoreMesh(core_axis_name="c", subcore_axis_name="s")

_GDN = lax.GatherDimensionNumbers(offset_dims=(), collapsed_slice_dims=(0,),
                                  start_index_map=(0,))


@functools.partial(
    pl.kernel,
    out_type=jax.ShapeDtypeStruct((NP, D), jnp.float32),
    mesh=_mesh,
    compiler_params=pltpu.CompilerParams(use_tc_tiling_on_sc=False),
    scratch_types=[
        pltpu.VMEM_SHARED((TAB, D), jnp.float32),
        pltpu.VMEM((NIDX, CH), jnp.int32),
        pltpu.VMEM((NIDX, CH), jnp.int32),
        pltpu.VMEM((NIDX, CH), jnp.float32),
        pltpu.VMEM((NGB, CH, D), jnp.float32),
        pltpu.VMEM((ZB, D), jnp.float32),
        [pltpu.SemaphoreType.DMA] * NIDX,
        [pltpu.SemaphoreType.DMA] * NGB,
        [pltpu.SemaphoreType.DMA] * NGB,
    ],
)
def _hop(agg_hbm, rows_hbm, cols_hbm, vals_hbm, out_hbm,
         table, colv, rowv, valv, gath, zbuf, isem, gsem, ssem):
    c = lax.axis_index("c")
    s = lax.axis_index("s")

    z16 = jnp.zeros((16,), jnp.float32)

    def _zrow(i, carry):
        for j in range(D // 16):
            zbuf[i, pl.ds(j * 16, 16)] = z16
        return carry

    lax.fori_loop(0, ZB, _zrow, 0)

    zbase = s * STRIPE
    nfull = STRIPE // ZB
    rem = STRIPE - nfull * ZB

    for q in range(nfull):
        pltpu.async_copy(zbuf, table.at[pl.ds(zbase + q * ZB, ZB)], gsem[0])
    if rem:
        pltpu.async_copy(zbuf.at[pl.ds(0, rem)],
                         table.at[pl.ds(zbase + nfull * ZB, rem)], gsem[0])
    for q in range(nfull):
        pltpu.make_async_copy(zbuf, table.at[pl.ds(zbase, ZB)], gsem[0]).wait()
    if rem:
        pltpu.make_async_copy(zbuf.at[pl.ds(0, rem)],
                              table.at[pl.ds(zbase, rem)], gsem[0]).wait()
    plsc.subcore_barrier()

    row_base = c * N_U
    e0 = s * ECS

    def p1(i, k):
        o = e0 + i * CH
        pltpu.async_copy(rows_hbm.at[pl.ds(o, CH)], rowv.at[k], isem[k])
        pltpu.async_copy(cols_hbm.at[pl.ds(o, CH)], colv.at[k], isem[k])
        pltpu.async_copy(vals_hbm.at[pl.ds(o, CH)], valv.at[k], isem[k])

    def p2(i, k8, k4, sswait):
        o = e0 + i * CH
        pltpu.make_async_copy(rows_hbm.at[pl.ds(o, CH)], rowv.at[k8],
                              isem[k8]).wait()
        pltpu.make_async_copy(cols_hbm.at[pl.ds(o, CH)], colv.at[k8],
                              isem[k8]).wait()
        pltpu.make_async_copy(vals_hbm.at[pl.ds(o, CH)], valv.at[k8],
                              isem[k8]).wait()
        for g in range(CH // 16):
            sl = pl.ds(g * 16, 16)
            cc = colv[k8, sl]
            colv[k8, sl] = jnp.where(cc >= N_U, cc + (TAB - N_U), cc)
        if sswait:
            pltpu.make_async_copy(gath.at[k4], table.at[rowv.at[k8]],
                                  ssem[k4]).wait()
        pltpu.async_copy(agg_hbm.at[colv.at[k8]], gath.at[k4], gsem[k4])

    def p3(i, k8, k4):
        pltpu.make_async_copy(agg_hbm.at[colv.at[k8]], gath.at[k4],
                              gsem[k4]).wait()

        @plsc.parallel_loop(0, CH // 16)
        def _mgrp(g):
            sl = pl.ds(g * 16, 16)
            rl = rowv[k8, sl] - row_base
            ok = (rl >= 0) & (rl < N_U)
            rowv[k8, sl] = jnp.where(ok, rl, 0)
            vv = jnp.where(ok, valv[k8, sl], 0.0)
            for l in range(16):
                vb = lax.gather(vv, jnp.full((16, 1), l, jnp.int32), _GDN, (1,),
                                mode=lax.GatherScatterMode.PROMISE_IN_BOUNDS)
                e = g * 16 + l
                for j in range(D // 16):
                    slj = pl.ds(j * 16, 16)
                    gath[k4, e, slj] = gath[k4, e, slj] * vb
        pltpu.async_copy(gath.at[k4], table.at[rowv.at[k8]], ssem[k4],
                         add=True)

    def step(i, kk):
        p2(i + 1, (kk + 1) % NIDX, (kk + 1) % NGB, True)
        p3(i, kk, kk % NGB)
        p1(i + 2, (kk + 2) % NIDX)

    p1(0, 0)
    p1(1, 1)
    p2(0, 0, 0, False)
    for i in range(NIDX):
        p2(i + 1, (i + 1) % NIDX, (i + 1) % NGB, (i + 1) >= NGB)
        p3(i, i, i % NGB)
        p1(i + 2, (i + 2) % NIDX)

    def _octet(t, carry):
        ib = t * NIDX
        for kk in range(NIDX):
            step(ib + kk, kk)
        return carry

    lax.fori_loop(1, NCHUNK // NIDX - 1, _octet, 0)

    base = NCHUNK - NIDX
    for kk in range(NIDX):
        i = base + kk
        if i + 1 < NCHUNK:
            p2(i + 1, (kk + 1) % NIDX, (kk + 1) % NGB, True)
        p3(i, kk, kk % NGB)
        if i + 2 < NCHUNK:
            p1(i + 2, (kk + 2) % NIDX)
    for k in range(NGB):
        k8 = (NCHUNK - NGB + k) % NIDX
        pltpu.make_async_copy(gath.at[k], table.at[rowv.at[k8]],
                              ssem[k]).wait()
    plsc.subcore_barrier()

    pltpu.sync_copy(table.at[pl.ds(zbase, STRIPE)],
                    out_hbm.at[pl.ds(c * TAB + zbase, STRIPE)])


_tail_out = [jax.ShapeDtypeStruct((B, D), jnp.float32)] * 6


@functools.partial(
    pl.kernel,
    out_type=_tail_out,
    mesh=_mesh,
    compiler_params=pltpu.CompilerParams(use_tc_tiling_on_sc=False),
    scratch_types=[
        pltpu.VMEM((CH,), jnp.int32),
        pltpu.VMEM((CH, D), jnp.float32),
        pltpu.VMEM((CH, D), jnp.float32),
        pltpu.SemaphoreType.DMA,
    ],
)
def _tail(t0, t1, t2, t3, uid, pid, nid,
          ue_o, pe_o, ne_o, u0_o, p0_o, n0_o,
          idxv, gath, acc, sem):
    c = lax.axis_index("c")
    s = lax.axis_index("s")
    w = s * NCORES + c
    base = w * CH

    for idx_hbm, mean_o, h0_o in ((uid, ue_o, u0_o),
                                  (pid, pe_o, p0_o),
                                  (nid, ne_o, n0_o)):
        pltpu.sync_copy(idx_hbm.at[pl.ds(base, CH)], idxv)
        pltpu.async_copy(t0.at[idxv], gath, sem).wait()
        pltpu.sync_copy(gath, h0_o.at[pl.ds(base, CH)])

        def _cp(e, carry):
            for j in range(D // 16):
                sl = pl.ds(j * 16, 16)
                acc[e, sl] = gath[e, sl]
            return carry

        lax.fori_loop(0, CH, _cp, 0)

        for t in (t1, t2, t3):
            pltpu.async_copy(t.at[idxv], gath, sem).wait()

            def _acc(e, carry):
                for j in range(D // 16):
                    sl = pl.ds(j * 16, 16)
                    acc[e, sl] = acc[e, sl] + gath[e, sl]
                return carry

            lax.fori_loop(0, CH, _acc, 0)

        quarter = jnp.full((16,), 0.25, jnp.float32)

        def _scale(e, carry):
            for j in range(D // 16):
                sl = pl.ds(j * 16, 16)
                acc[e, sl] = acc[e, sl] * quarter
            return carry

        lax.fori_loop(0, CH, _scale, 0)
        pltpu.sync_copy(acc, mean_o.at[pl.ds(base, CH)])


def _loss_body(ue_ref, pe_ref, ne_ref, u0_ref, p0_ref, n0_ref, out_ref):
    ue = ue_ref[...]
    pos_s = jnp.sum(ue * pe_ref[...], axis=1, keepdims=True)
    neg_s = jnp.sum(ue * ne_ref[...], axis=1, keepdims=True)
    mf = jnp.mean(jnp.log(1.0 + jnp.exp(neg_s - pos_s)))
    reg = (jnp.sum(u0_ref[...] ** 2) + jnp.sum(p0_ref[...] ** 2)
           + jnp.sum(n0_ref[...] ** 2)) * 0.5
    emb = DECAY * reg / B
    lanes = lax.broadcasted_iota(jnp.int32, (1, 128), 1)
    out_ref[...] = jnp.where(
        lanes == 0, mf + emb,
        jnp.where(lanes == 1, mf, jnp.where(lanes == 2, emb, 0.0)))


_loss = pl.pallas_call(
    _loss_body,
    out_shape=jax.ShapeDtypeStruct((1, 128), jnp.float32),
)


def kernel(user_embed, item_embed, edge_values, edge_index, users,
           pos_items, neg_items, cur_epoch):
    zpad = jnp.zeros((TAB - N_U, D), jnp.float32)
    all0 = jnp.concatenate([user_embed, zpad, item_embed, zpad], axis=0)
    pad = EP - E
    rows_p = jnp.pad(edge_index[0], (0, pad))
    cols_p = jnp.pad(edge_index[1], (0, pad))
    vals_p = jnp.pad(edge_values, (0, pad))

    a1 = _hop(all0, rows_p, cols_p, vals_p)
    a2 = _hop(a1, rows_p, cols_p, vals_p)
    a3 = _hop(a2, rows_p, cols_p, vals_p)

    uid = users
    pid = pos_items + TAB
    nid = neg_items[:, 0] + TAB
    ue, pe, ne, u0, p0, n0 = _tail(all0, a1, a2, a3, uid, pid, nid)

    out = _loss(ue, pe, ne, u0, p0, n0)
    return (out[0, 0], out[0, 1], out[0, 2])


# speedup vs baseline: 5.8465x; 1.0046x over previous
import functools

import jax
import jax.numpy as jnp
import numpy as np
from jax import lax
from jax.experimental import pallas as pl
from jax.experimental.pallas import tpu as pltpu
from jax.experimental.pallas import tpu_sc as plsc

N_U = 25000
N_I = 25000
N = N_U + N_I
D = 64
E = 800000
HOPS = 3
B = 4096
DECAY = 1e-4

NCORES = 2
NSUB = 16
CH = 128
NCHUNK = 392
EP = NSUB * NCHUNK * CH
ECS = EP // NSUB
NIDX = 8
NGB = 2
TAB = 25088
NP = 2 * TAB
STRIPE = TAB // NSUB
ZB = 32

_mesh = plsc.VectorSubcoreMesh(core_axis_name="c", subcore_axis_name="s")

_GDN = lax.GatherDimensionNumbers(offset_dims=(), collapsed_slice_dims=(0,),
                                  start_index_map=(0,))


@functools.partial(
    pl.kernel,
    out_type=[
        jax.ShapeDtypeStruct((2 * EP,), jnp.int32),
        jax.ShapeDtypeStruct((2 * EP,), jnp.int32),
        jax.ShapeDtypeStruct((2 * EP,), jnp.float32),
        jax.ShapeDtypeStruct((NCORES * NSUB * 16,), jnp.int32),
    ],
    mesh=_mesh,
    compiler_params=pltpu.CompilerParams(use_tc_tiling_on_sc=False,
                                         needs_layout_passes=False),
    scratch_types=[
        pltpu.VMEM((NIDX, CH), jnp.int32),
        pltpu.VMEM((NIDX, CH), jnp.int32),
        pltpu.VMEM((NIDX, CH), jnp.float32),
        pltpu.VMEM((2 * CH + 16, ), jnp.int32),
        pltpu.VMEM((2 * CH + 16, ), jnp.int32),
        pltpu.VMEM((2 * CH + 16, ), jnp.float32),
        pltpu.VMEM((CH,), jnp.int32),
        pltpu.VMEM((CH,), jnp.int32),
        pltpu.VMEM((CH,), jnp.float32),
        pltpu.VMEM((CH,), jnp.int32),
        pltpu.VMEM((CH,), jnp.int32),
        pltpu.VMEM((CH,), jnp.float32),
        pltpu.VMEM((CH,), jnp.int32),
        pltpu.VMEM((CH,), jnp.float32),
        pltpu.VMEM((16,), jnp.int32),
        [pltpu.SemaphoreType.DMA] * NIDX,
        [pltpu.SemaphoreType.DMA] * 2,
    ],
)
def _partition(rows_hbm, cols_hbm, vals_hbm, ccol, crow, cval, counts,
               colv, rowv, valv, pcol, prow, pval,
               f0c, f0r, f0v, f1c, f1r, f1v, zi, zf, cbuf, isem, fsem):
    c = lax.axis_index("c")
    s = lax.axis_index("s")
    e0 = s * ECS
    wb = (c * NSUB + s) * ECS
    row_base = c * N_U
    fc = (f0c, f1c)
    fr = (f0r, f1r)
    fv = (f0v, f1v)

    z16i = jnp.zeros((16,), jnp.int32)
    z16f = jnp.zeros((16,), jnp.float32)
    for g in range(CH // 16):
        sl = pl.ds(g * 16, 16)
        zi[sl] = z16i
        zf[sl] = z16f

    def p1(i, k):
        o = e0 + i * CH
        pltpu.async_copy(rows_hbm.at[pl.ds(o, CH)], rowv.at[k], isem[k])
        pltpu.async_copy(cols_hbm.at[pl.ds(o, CH)], colv.at[k], isem[k])
        pltpu.async_copy(vals_hbm.at[pl.ds(o, CH)], valv.at[k], isem[k])

    def pwait(i, k):
        o = e0 + i * CH
        pltpu.make_async_copy(rows_hbm.at[pl.ds(o, CH)], rowv.at[k],
                              isem[k]).wait()
        pltpu.make_async_copy(cols_hbm.at[pl.ds(o, CH)], colv.at[k],
                              isem[k]).wait()
        pltpu.make_async_copy(vals_hbm.at[pl.ds(o, CH)], valv.at[k],
                              isem[k]).wait()

    def fsem_wait(q):
        pltpu.make_async_copy(fc[q], ccol.at[pl.ds(wb, CH)], fsem[q]).wait()
        pltpu.make_async_copy(fr[q], crow.at[pl.ds(wb, CH)], fsem[q]).wait()
        pltpu.make_async_copy(fv[q], cval.at[pl.ds(wb, CH)], fsem[q]).wait()

    def process(i, kk, st):
        cnt, nf, pd0, pd1 = st
        pwait(i, kk)
        for g in range(CH // 16):
            sl = pl.ds(g * 16, 16)
            rl = rowv[kk, sl] - row_base
            ok = (rl >= 0) & (rl < N_U)
            cc = colv[kk, sl]
            cc = jnp.where(cc >= N_U, cc + (TAB - N_U), cc)
            pref = plsc.cumsum(jnp.where(ok, 1, 0))
            dest = jnp.where(ok, cnt + pref - 1, 2 * CH)
            plsc.store_scatter(prow, [dest], rl)
            plsc.store_scatter(pcol, [dest], cc)
            plsc.store_scatter(pval, [dest], valv[kk, sl])
            cnt = cnt + pref[15]
        fire = cnt >= CH
        q = kk % 2
        pdq = pd0 if q == 0 else pd1

        @pl.when(fire)
        def _():
            @pl.when(pdq == 1)
            def _():
                fsem_wait(q)
            for g in range(CH // 16):
                sl = pl.ds(g * 16, 16)
                fc[q][sl] = pcol[sl]
                fr[q][sl] = prow[sl]
                fv[q][sl] = pval[sl]
            o = wb + nf * CH
            pltpu.async_copy(fc[q], ccol.at[pl.ds(o, CH)], fsem[q])
            pltpu.async_copy(fr[q], crow.at[pl.ds(o, CH)], fsem[q])
            pltpu.async_copy(fv[q], cval.at[pl.ds(o, CH)], fsem[q])
            for g in range(CH // 16):
                sl = pl.ds(g * 16, 16)
                sh = pl.ds(CH + g * 16, 16)
                pcol[sl] = pcol[sh]
                prow[sl] = prow[sh]
                pval[sl] = pval[sh]

        cnt = jnp.where(fire, cnt - CH, cnt)
        nf = jnp.where(fire, nf + 1, nf)
        one = jnp.int32(1)
        if q == 0:
            pd0 = jnp.where(fire, one, pd0)
        else:
            pd1 = jnp.where(fire, one, pd1)
        return (cnt, nf, pd0, pd1)

    st = (jnp.int32(0), jnp.int32(0), jnp.int32(0), jnp.int32(0))
    p1(0, 0)
    p1(1, 1)
    for i in range(NIDX):
        st = process(i, i, st)
        p1(i + 2, (i + 2) % NIDX)

    def _octet(t, st):
        ib = t * NIDX
        for kk in range(NIDX):
            st = process(ib + kk, kk, st)
            p1(ib + kk + 2, (kk + 2) % NIDX)
        return st

    st = lax.fori_loop(1, NCHUNK // NIDX - 1, _octet, st)

    base = NCHUNK - NIDX
    for kk in range(NIDX):
        st = process(base + kk, kk, st)
        if kk < NIDX - 2:
            p1(base + kk + 2, (kk + 2) % NIDX)
    cnt, nf, pd0, pd1 = st

    @pl.when(pd0 == 1)
    def _():
        fsem_wait(0)

    @pl.when(pd1 == 1)
    def _():
        fsem_wait(1)

    for g in range(CH // 16):
        sl = pl.ds(g * 16, 16)
        pos = jnp.int32(g * 16) + lax.iota(jnp.int32, 16)
        keep = pos < cnt
        pcol[sl] = jnp.where(keep, pcol[sl], 0)
        prow[sl] = jnp.where(keep, prow[sl], 0)
        pval[sl] = jnp.where(keep, pval[sl], 0.0)

    @pl.when(cnt > 0)
    def _():
        o = wb + nf * CH
        pltpu.sync_copy(pcol.at[pl.ds(0, CH)], ccol.at[pl.ds(o, CH)])
        pltpu.sync_copy(prow.at[pl.ds(0, CH)], crow.at[pl.ds(o, CH)])
        pltpu.sync_copy(pval.at[pl.ds(0, CH)], cval.at[pl.ds(o, CH)])

    nf = jnp.where(cnt > 0, nf + 1, nf)
    tgt = jnp.maximum(jnp.int32(2 * NIDX), ((nf + NIDX - 1) // NIDX) * NIDX)

    def _zc(i, carry):
        o = wb + i * CH
        pltpu.sync_copy(zi, ccol.at[pl.ds(o, CH)])
        pltpu.sync_copy(zi, crow.at[pl.ds(o, CH)])
        pltpu.sync_copy(zf, cval.at[pl.ds(o, CH)])
        return carry

    lax.fori_loop(nf, tgt, _zc, 0)

    cbuf[pl.ds(0, 16)] = jnp.full((16,), 1, jnp.int32) * tgt
    pltpu.sync_copy(cbuf, counts.at[pl.ds((c * NSUB + s) * 16, 16)])


@functools.partial(
    pl.kernel,
    out_type=jax.ShapeDtypeStruct((NP, D), jnp.float32),
    mesh=_mesh,
    compiler_params=pltpu.CompilerParams(use_tc_tiling_on_sc=False),
    scratch_types=[
        pltpu.VMEM_SHARED((TAB, D), jnp.float32),
        pltpu.VMEM((NIDX, CH), jnp.int32),
        pltpu.VMEM((NIDX, CH), jnp.int32),
        pltpu.VMEM((NIDX, CH), jnp.float32),
        pltpu.VMEM((NGB, CH, D), jnp.float32),
        pltpu.VMEM((ZB, D), jnp.float32),
        pltpu.VMEM((16,), jnp.int32),
        [pltpu.SemaphoreType.DMA] * NIDX,
        [pltpu.SemaphoreType.DMA] * NGB,
        [pltpu.SemaphoreType.DMA] * NGB,
    ],
)
def _hop(agg_hbm, ccol_hbm, crow_hbm, cval_hbm, cnt_hbm, out_hbm,
         table, colv, rowv, valv, gath, zbuf, cntv, isem, gsem, ssem):
    c = lax.axis_index("c")
    s = lax.axis_index("s")

    z16 = jnp.zeros((16,), jnp.float32)

    def _zrow(i, carry):
        for j in range(D // 16):
            zbuf[i, pl.ds(j * 16, 16)] = z16
        return carry

    lax.fori_loop(0, ZB, _zrow, 0)

    zbase = s * STRIPE
    nfull = STRIPE // ZB
    rem = STRIPE - nfull * ZB

    for q in range(nfull):
        pltpu.async_copy(zbuf, table.at[pl.ds(zbase + q * ZB, ZB)], gsem[0])
    if rem:
        pltpu.async_copy(zbuf.at[pl.ds(0, rem)],
                         table.at[pl.ds(zbase + nfull * ZB, rem)], gsem[0])
    for q in range(nfull):
        pltpu.make_async_copy(zbuf, table.at[pl.ds(zbase, ZB)], gsem[0]).wait()
    if rem:
        pltpu.make_async_copy(zbuf.at[pl.ds(0, rem)],
                              table.at[pl.ds(zbase, rem)], gsem[0]).wait()
    plsc.subcore_barrier()

    wb = (c * NSUB + s) * ECS
    pltpu.sync_copy(cnt_hbm.at[pl.ds((c * NSUB + s) * 16, 16)], cntv)
    nq = cntv[pl.ds(0, 16)][0]

    def p1(i, k):
        o = wb + i * CH
        pltpu.async_copy(crow_hbm.at[pl.ds(o, CH)], rowv.at[k], isem[k])
        pltpu.async_copy(ccol_hbm.at[pl.ds(o, CH)], colv.at[k], isem[k])
        pltpu.async_copy(cval_hbm.at[pl.ds(o, CH)], valv.at[k], isem[k])

    def p2(i, k8, k4, sswait):
        o = wb + i * CH
        pltpu.make_async_copy(crow_hbm.at[pl.ds(o, CH)], rowv.at[k8],
                              isem[k8]).wait()
        pltpu.make_async_copy(ccol_hbm.at[pl.ds(o, CH)], colv.at[k8],
                              isem[k8]).wait()
        pltpu.make_async_copy(cval_hbm.at[pl.ds(o, CH)], valv.at[k8],
                              isem[k8]).wait()
        if sswait:
            pltpu.make_async_copy(gath.at[k4], table.at[rowv.at[k8]],
                                  ssem[k4]).wait()
        pltpu.async_copy(agg_hbm.at[colv.at[k8]], gath.at[k4], gsem[k4])

    def p3(i, k8, k4):
        pltpu.make_async_copy(agg_hbm.at[colv.at[k8]], gath.at[k4],
                              gsem[k4]).wait()

        @plsc.parallel_loop(0, CH // 16)
        def _mgrp(g):
            sl = pl.ds(g * 16, 16)
            vv = valv[k8, sl]
            for l in range(16):
                vb = lax.gather(vv, jnp.full((16, 1), l, jnp.int32), _GDN, (1,),
                                mode=lax.GatherScatterMode.PROMISE_IN_BOUNDS)
                e = g * 16 + l
                for j in range(D // 16):
                    slj = pl.ds(j * 16, 16)
                    gath[k4, e, slj] = gath[k4, e, slj] * vb
        pltpu.async_copy(gath.at[k4], table.at[rowv.at[k8]], ssem[k4],
                         add=True)

    def step(i, kk):
        p2(i + 1, (kk + 1) % NIDX, (kk + 1) % NGB, True)
        p3(i, kk, kk % NGB)
        p1(i + 2, (kk + 2) % NIDX)

    p1(0, 0)
    p1(1, 1)
    p2(0, 0, 0, False)
    for i in range(NIDX):
        p2(i + 1, (i + 1) % NIDX, (i + 1) % NGB, (i + 1) >= NGB)
        p3(i, i, i % NGB)
        p1(i + 2, (i + 2) % NIDX)

    def _octet(t, carry):
        ib = t * NIDX
        for kk in range(NIDX):
            step(ib + kk, kk)
        return carry

    lax.fori_loop(1, nq // NIDX - 1, _octet, 0)

    base = nq - NIDX
    for kk in range(NIDX):
        i = base + kk
        if kk + 1 < NIDX:
            p2(i + 1, (kk + 1) % NIDX, (kk + 1) % NGB, True)
        p3(i, kk, kk % NGB)
        if kk + 2 < NIDX:
            p1(i + 2, (kk + 2) % NIDX)
    for k in range(NGB):
        pltpu.make_async_copy(gath.at[k], table.at[rowv.at[0]],
                              ssem[k]).wait()
    plsc.subcore_barrier()

    pltpu.sync_copy(table.at[pl.ds(zbase, STRIPE)],
                    out_hbm.at[pl.ds(c * TAB + zbase, STRIPE)])


_tail_out = [jax.ShapeDtypeStruct((B, D), jnp.float32)] * 6


@functools.partial(
    pl.kernel,
    out_type=_tail_out,
    mesh=_mesh,
    compiler_params=pltpu.CompilerParams(use_tc_tiling_on_sc=False),
    scratch_types=[
        pltpu.VMEM((CH,), jnp.int32),
        pltpu.VMEM((CH, D), jnp.float32),
        pltpu.VMEM((CH, D), jnp.float32),
        pltpu.SemaphoreType.DMA,
    ],
)
def _tail(t0, t1, t2, t3, uid, pid, nid,
          ue_o, pe_o, ne_o, u0_o, p0_o, n0_o,
          idxv, gath, acc, sem):
    c = lax.axis_index("c")
    s = lax.axis_index("s")
    w = s * NCORES + c
    base = w * CH

    for idx_hbm, mean_o, h0_o in ((uid, ue_o, u0_o),
                                  (pid, pe_o, p0_o),
                                  (nid, ne_o, n0_o)):
        pltpu.sync_copy(idx_hbm.at[pl.ds(base, CH)], idxv)
        pltpu.async_copy(t0.at[idxv], gath, sem).wait()
        pltpu.sync_copy(gath, h0_o.at[pl.ds(base, CH)])

        def _cp(e, carry):
            for j in range(D // 16):
                sl = pl.ds(j * 16, 16)
                acc[e, sl] = gath[e, sl]
            return carry

        lax.fori_loop(0, CH, _cp, 0)

        for t in (t1, t2, t3):
            pltpu.async_copy(t.at[idxv], gath, sem).wait()

            def _acc(e, carry):
                for j in range(D // 16):
                    sl = pl.ds(j * 16, 16)
                    acc[e, sl] = acc[e, sl] + gath[e, sl]
                return carry

            lax.fori_loop(0, CH, _acc, 0)

        quarter = jnp.full((16,), 0.25, jnp.float32)

        def _scale(e, carry):
            for j in range(D // 16):
                sl = pl.ds(j * 16, 16)
                acc[e, sl] = acc[e, sl] * quarter
            return carry

        lax.fori_loop(0, CH, _scale, 0)
        pltpu.sync_copy(acc, mean_o.at[pl.ds(base, CH)])


def _loss_body(ue_ref, pe_ref, ne_ref, u0_ref, p0_ref, n0_ref, out_ref):
    ue = ue_ref[...]
    pos_s = jnp.sum(ue * pe_ref[...], axis=1, keepdims=True)
    neg_s = jnp.sum(ue * ne_ref[...], axis=1, keepdims=True)
    mf = jnp.mean(jnp.log(1.0 + jnp.exp(neg_s - pos_s)))
    reg = (jnp.sum(u0_ref[...] ** 2) + jnp.sum(p0_ref[...] ** 2)
           + jnp.sum(n0_ref[...] ** 2)) * 0.5
    emb = DECAY * reg / B
    lanes = lax.broadcasted_iota(jnp.int32, (1, 128), 1)
    out_ref[...] = jnp.where(
        lanes == 0, mf + emb,
        jnp.where(lanes == 1, mf, jnp.where(lanes == 2, emb, 0.0)))


_loss = pl.pallas_call(
    _loss_body,
    out_shape=jax.ShapeDtypeStruct((1, 128), jnp.float32),
)


def kernel(user_embed, item_embed, edge_values, edge_index, users,
           pos_items, neg_items, cur_epoch):
    zpad = jnp.zeros((TAB - N_U, D), jnp.float32)
    all0 = jnp.concatenate([user_embed, zpad, item_embed, zpad], axis=0)
    pad = EP - E
    rows_p = jnp.pad(edge_index[0], (0, pad))
    cols_p = jnp.pad(edge_index[1], (0, pad))
    vals_p = jnp.pad(edge_values, (0, pad))

    ccol, crow, cval, counts = _partition(rows_p, cols_p, vals_p)
    a1 = _hop(all0, ccol, crow, cval, counts)
    a2 = _hop(a1, ccol, crow, cval, counts)
    a3 = _hop(a2, ccol, crow, cval, counts)

    uid = users
    pid = pos_items + TAB
    nid = neg_items[:, 0] + TAB
    ue, pe, ne, u0, p0, n0 = _tail(all0, a1, a2, a3, uid, pid, nid)

    out = _loss(ue, pe, ne, u0, p0, n0)
    return (out[0, 0], out[0, 1], out[0, 2])
